# baseline (speedup 1.0000x reference)
import functools

import jax
import jax.numpy as jnp
from jax import lax
from jax.experimental import pallas as pl
from jax.experimental.pallas import tpu as pltpu
from jax.experimental.pallas import tpu_sc as plsc

BLK = 1024


def _sc_gather(feat, idx32):
    b = idx32.shape[0]
    d = feat.shape[1]
    window = 128
    mesh = plsc.VectorSubcoreMesh(core_axis_name="core", subcore_axis_name="subcore")
    indices = idx32.reshape((1, b))

    @functools.partial(
        pl.kernel,
        out_type=jax.ShapeDtypeStruct((b, d), feat.dtype),
        mesh=mesh,
    )
    def kern(x_hbm, i_hbm, o_hbm):
        def body(i_vmem, o_vmem):
            pltpu.sync_copy(x_hbm.at[i_vmem.at[0]], o_vmem)

        pltpu.emit_pipeline(
            body,
            grid=(b // window,),
            in_specs=[pl.BlockSpec((1, window), index_map=lambda i: (0, i))],
            out_specs=[pl.BlockSpec((window, d), index_map=lambda i: (i, 0))],
            core_axis_name="subcore",
            dimension_semantics=(pltpu.PARALLEL,),
        )(i_hbm, o_hbm)

    return kern(feat, indices)


def _hi_lo(x):
    hi = x.astype(jnp.bfloat16)
    lo = (x - hi.astype(jnp.float32)).astype(jnp.bfloat16)
    return hi, lo


def _tc_body(feat_ref, seg_ref, gat_ref, wu_ref, wv_ref, bv_ref, we_ref, pw_ref,
             out_ref, fvh_s, fvl_s, wuh_s, wul_s, eexp_s, den_s, *, n, b, nb):
    p = pl.program_id(0)
    i = pl.program_id(1)
    f32 = jnp.float32
    bf16 = jnp.bfloat16

    def dot_nt(x, y):
        return lax.dot_general(x, y, (((1,), (1,)), ((), ())),
                               preferred_element_type=f32)

    @pl.when(jnp.logical_and(p == 0, i == 0))
    def _prologue():
        fv = lax.dot_general(gat_ref[...], wv_ref[...], (((1,), (1,)), ((), ())),
                             preferred_element_type=f32) + bv_ref[...]
        fvh_s[...], fvl_s[...] = _hi_lo(fv)
        wuh_s[...], wul_s[...] = _hi_lo(wu_ref[...])
        den_s[...] = jnp.zeros_like(den_s)

    seg = seg_ref[...]
    oh = (seg == lax.broadcasted_iota(jnp.int32, (BLK, b), 1)).astype(bf16)
    valid_row = (i * BLK + lax.broadcasted_iota(jnp.int32, (1, BLK), 1)) < n

    @pl.when(p == 0)
    def _phase0():
        fh, fl = _hi_lo(feat_ref[...])
        u = (dot_nt(fh, wuh_s[...]) + dot_nt(fh, wul_s[...])
             + dot_nt(fl, wuh_s[...]))
        fvb = (lax.dot_general(oh, fvh_s[...], (((1,), (0,)), ((), ())),
                               preferred_element_type=f32)
               + lax.dot_general(oh, fvl_s[...], (((1,), (0,)), ((), ())),
                                 preferred_element_type=f32))
        s = jax.nn.sigmoid(u + fvb)
        e_row = dot_nt(we_ref[...], s)
        eexp = jnp.where(valid_row, jnp.exp(e_row), 0.0)
        eexp_s[pl.ds(i, 1), :] = eexp
        eh, el = _hi_lo(eexp)
        den_s[...] += (lax.dot_general(eh, oh, (((1,), (0,)), ((), ())),
                                       preferred_element_type=f32)
                       + lax.dot_general(el, oh, (((1,), (0,)), ((), ())),
                                         preferred_element_type=f32))

    @pl.when(p == 1)
    def _phase1():
        eexp = eexp_s[pl.ds(i, 1), :]
        dh, dl = _hi_lo(den_s[...])
        dg = dot_nt(dh, oh) + dot_nt(dl, oh)
        alpha_col = jnp.transpose(eexp / dg, (1, 0))
        valid_col = (i * BLK + lax.broadcasted_iota(jnp.int32, (BLK, 1), 0)) < n
        featn = jnp.where(valid_col, feat_ref[...] * alpha_col, 0.0)
        contrib = lax.dot_general(oh, featn.astype(bf16),
                                  (((0,), (0,)), ((), ())),
                                  preferred_element_type=f32)

        @pl.when(i == 0)
        def _():
            out_ref[...] = contrib

        @pl.when(i > 0)
        def _():
            out_ref[...] += contrib

        @pl.when(i == nb - 1)
        def _():
            acc = out_ref[...]
            out_ref[...] = jnp.where(acc > 0, acc, pw_ref[...] * acc)


def _tc_main(feat, seg_pad, gathered, W_u, W_v, b_v, W_e, prelu_w):
    n, d = feat.shape
    h = W_u.shape[0]
    b = gathered.shape[0]
    nb = seg_pad.shape[0] // BLK
    nb_pad = ((nb + 7) // 8) * 8

    grid = (2, nb)
    body = functools.partial(_tc_body, n=n, b=b, nb=nb)
    return pl.pallas_call(
        body,
        grid=grid,
        in_specs=[
            pl.BlockSpec((BLK, d), lambda p, i: (i, 0)),
            pl.BlockSpec((BLK, 1), lambda p, i: (i, 0)),
            pl.BlockSpec((b, d), lambda p, i: (0, 0)),
            pl.BlockSpec((h, d), lambda p, i: (0, 0)),
            pl.BlockSpec((h, d), lambda p, i: (0, 0)),
            pl.BlockSpec((1, h), lambda p, i: (0, 0)),
            pl.BlockSpec((1, h), lambda p, i: (0, 0)),
            pl.BlockSpec((1, d), lambda p, i: (0, 0)),
        ],
        out_specs=pl.BlockSpec((b, d), lambda p, i: (0, 0)),
        out_shape=jax.ShapeDtypeStruct((b, d), jnp.float32),
        scratch_shapes=[
            pltpu.VMEM((b, h), jnp.bfloat16),
            pltpu.VMEM((b, h), jnp.bfloat16),
            pltpu.VMEM((h, d), jnp.bfloat16),
            pltpu.VMEM((h, d), jnp.bfloat16),
            pltpu.VMEM((nb_pad, BLK), jnp.float32),
            pltpu.VMEM((1, b), jnp.float32),
        ],
        compiler_params=pltpu.CompilerParams(
            dimension_semantics=("arbitrary", "arbitrary"),
        ),
    )(feat, seg_pad, gathered, W_u, W_v, b_v, W_e, prelu_w)


def kernel(feat, segment_ids, last_nodes, W_u, W_v, b_v, W_e, prelu_w):
    n, d = feat.shape
    h = W_u.shape[0]
    b = last_nodes.shape[0]
    nb = -(-n // BLK)
    np_ = nb * BLK

    seg32 = segment_ids.astype(jnp.int32)
    seg_pad = jnp.full((np_,), b, jnp.int32).at[:n].set(seg32).reshape(np_, 1)
    idx32 = last_nodes.astype(jnp.int32)

    gathered = _sc_gather(feat, idx32)
    return _tc_main(feat, seg_pad, gathered,
                    W_u, W_v,
                    b_v.reshape(1, h).astype(jnp.float32),
                    W_e, prelu_w.reshape(1, d).astype(jnp.float32))


# speedup vs baseline: 1.4179x; 1.4179x over previous
import functools

import jax
import jax.numpy as jnp
from jax import lax
from jax.experimental import pallas as pl
from jax.experimental.pallas import tpu as pltpu
from jax.experimental.pallas import tpu_sc as plsc

BLK = 512


def _sc_gather(feat, idx32):
    b = idx32.shape[0]
    d = feat.shape[1]
    window = 128
    mesh = plsc.VectorSubcoreMesh(core_axis_name="core", subcore_axis_name="subcore")
    indices = idx32.reshape((1, b))

    @functools.partial(
        pl.kernel,
        out_type=jax.ShapeDtypeStruct((b, d), feat.dtype),
        mesh=mesh,
    )
    def kern(x_hbm, i_hbm, o_hbm):
        def body(i_vmem, o_vmem):
            pltpu.sync_copy(x_hbm.at[i_vmem.at[0]], o_vmem)

        pltpu.emit_pipeline(
            body,
            grid=(b // window,),
            in_specs=[pl.BlockSpec((1, window), index_map=lambda i: (0, i))],
            out_specs=[pl.BlockSpec((window, d), index_map=lambda i: (i, 0))],
            core_axis_name="subcore",
            dimension_semantics=(pltpu.PARALLEL,),
        )(i_hbm, o_hbm)

    return kern(feat, indices)


def _tc_body(feat_ref, seg_ref, gat_ref, wu_ref, wv_ref, bv_ref, we_ref, pw_ref,
             out_ref, fv_s, eexp_s, den_s, *, n, b, nb):
    p = pl.program_id(0)
    i = pl.program_id(1)
    f32 = jnp.float32

    @pl.when(jnp.logical_and(p == 0, i == 0))
    def _prologue():
        fv = lax.dot_general(gat_ref[...], wv_ref[...], (((1,), (1,)), ((), ())),
                             preferred_element_type=f32)
        fv_s[...] = fv + bv_ref[...]
        den_s[...] = jnp.zeros_like(den_s)

    seg = seg_ref[...]
    iota_b = lax.broadcasted_iota(jnp.int32, (BLK, b), 1)
    valid_row = (i * BLK + lax.broadcasted_iota(jnp.int32, (1, BLK), 1)) < n

    @pl.when(p == 0)
    def _phase0():
        onehot = (seg == iota_b).astype(f32)
        u = lax.dot_general(feat_ref[...], wu_ref[...], (((1,), (1,)), ((), ())),
                            preferred_element_type=f32)
        fvb = lax.dot_general(onehot, fv_s[...], (((1,), (0,)), ((), ())),
                              preferred_element_type=f32)
        s = jax.nn.sigmoid(u + fvb)
        e_row = lax.dot_general(we_ref[...], s, (((1,), (1,)), ((), ())),
                                preferred_element_type=f32)
        eexp = jnp.where(valid_row, jnp.exp(e_row), 0.0)
        eexp_s[pl.ds(i, 1), :] = eexp
        den_s[...] += lax.dot_general(eexp, onehot, (((1,), (0,)), ((), ())),
                                      preferred_element_type=f32)

    @pl.when(p == 1)
    def _phase1():
        oh_bf = (seg == iota_b).astype(jnp.bfloat16)
        eexp_col = jnp.transpose(eexp_s[pl.ds(i, 1), :], (1, 0))
        valid_col = (i * BLK + lax.broadcasted_iota(jnp.int32, (BLK, 1), 0)) < n
        featn = jnp.where(valid_col, feat_ref[...] * eexp_col, 0.0)
        contrib = lax.dot_general(oh_bf, featn.astype(jnp.bfloat16),
                                  (((0,), (0,)), ((), ())),
                                  preferred_element_type=f32)

        @pl.when(i == 0)
        def _():
            out_ref[...] = contrib

        @pl.when(i > 0)
        def _():
            out_ref[...] += contrib

        @pl.when(i == nb - 1)
        def _():
            den = den_s[...]
            inv_col = jnp.transpose(jnp.where(den > 0, 1.0 / den, 0.0), (1, 0))
            acc = out_ref[...] * inv_col
            out_ref[...] = jnp.where(acc > 0, acc, pw_ref[...] * acc)


def _tc_main(feat, seg_pad, gathered, W_u, W_v, b_v, W_e, prelu_w):
    n, d = feat.shape
    h = W_u.shape[0]
    b = gathered.shape[0]
    nb = seg_pad.shape[0] // BLK
    nb_pad = ((nb + 7) // 8) * 8

    grid = (2, nb)
    body = functools.partial(_tc_body, n=n, b=b, nb=nb)
    return pl.pallas_call(
        body,
        grid=grid,
        in_specs=[
            pl.BlockSpec((BLK, d), lambda p, i: (i, 0)),
            pl.BlockSpec((BLK, 1), lambda p, i: (i, 0)),
            pl.BlockSpec((b, d), lambda p, i: (0, 0)),
            pl.BlockSpec((h, d), lambda p, i: (0, 0)),
            pl.BlockSpec((h, d), lambda p, i: (0, 0)),
            pl.BlockSpec((1, h), lambda p, i: (0, 0)),
            pl.BlockSpec((1, h), lambda p, i: (0, 0)),
            pl.BlockSpec((1, d), lambda p, i: (0, 0)),
        ],
        out_specs=pl.BlockSpec((b, d), lambda p, i: (0, 0)),
        out_shape=jax.ShapeDtypeStruct((b, d), jnp.float32),
        scratch_shapes=[
            pltpu.VMEM((b, h), jnp.float32),
            pltpu.VMEM((nb_pad, BLK), jnp.float32),
            pltpu.VMEM((1, b), jnp.float32),
        ],
        compiler_params=pltpu.CompilerParams(
            dimension_semantics=("arbitrary", "arbitrary"),
        ),
    )(feat, seg_pad, gathered, W_u, W_v, b_v, W_e, prelu_w)


def kernel(feat, segment_ids, last_nodes, W_u, W_v, b_v, W_e, prelu_w):
    n, d = feat.shape
    h = W_u.shape[0]
    b = last_nodes.shape[0]
    nb = -(-n // BLK)
    np_ = nb * BLK

    seg32 = segment_ids.astype(jnp.int32)
    seg_pad = jnp.full((np_,), b, jnp.int32).at[:n].set(seg32).reshape(np_, 1)
    idx32 = last_nodes.astype(jnp.int32)

    gathered = _sc_gather(feat, idx32)
    return _tc_main(feat, seg_pad, gathered,
                    W_u, W_v,
                    b_v.reshape(1, h).astype(jnp.float32),
                    W_e, prelu_w.reshape(1, d).astype(jnp.float32))


# speedup vs baseline: 1.7656x; 1.2452x over previous
import functools

import jax
import jax.numpy as jnp
from jax import lax
from jax.experimental import pallas as pl
from jax.experimental.pallas import tpu as pltpu
from jax.experimental.pallas import tpu_sc as plsc

BLK = 1024


def _sc_gather(feat, idx32):
    b = idx32.shape[0]
    d = feat.shape[1]
    window = 128
    mesh = plsc.VectorSubcoreMesh(core_axis_name="core", subcore_axis_name="subcore")
    indices = idx32.reshape((1, b))

    @functools.partial(
        pl.kernel,
        out_type=jax.ShapeDtypeStruct((b, d), feat.dtype),
        mesh=mesh,
    )
    def kern(x_hbm, i_hbm, o_hbm):
        def body(i_vmem, o_vmem):
            pltpu.sync_copy(x_hbm.at[i_vmem.at[0]], o_vmem)

        pltpu.emit_pipeline(
            body,
            grid=(b // window,),
            in_specs=[pl.BlockSpec((1, window), index_map=lambda i: (0, i))],
            out_specs=[pl.BlockSpec((window, d), index_map=lambda i: (i, 0))],
            core_axis_name="subcore",
            dimension_semantics=(pltpu.PARALLEL,),
        )(i_hbm, o_hbm)

    return kern(feat, indices)


def _tc_body(feat_ref, seg_ref, gat_ref, wu_ref, wv_ref, bv_ref, we_ref, pw_ref,
             out_ref, fv_s, eexp_s, den_s, *, n, b, nb):
    p = pl.program_id(0)
    i = pl.program_id(1)
    f32 = jnp.float32

    @pl.when(jnp.logical_and(p == 0, i == 0))
    def _prologue():
        fv = lax.dot_general(gat_ref[...], wv_ref[...], (((1,), (1,)), ((), ())),
                             preferred_element_type=f32)
        fv_s[...] = fv + bv_ref[...]
        den_s[...] = jnp.zeros_like(den_s)

    seg = seg_ref[...]
    iota_b = lax.broadcasted_iota(jnp.int32, (BLK, b), 1)
    valid_row = (i * BLK + lax.broadcasted_iota(jnp.int32, (1, BLK), 1)) < n

    @pl.when(p == 0)
    def _phase0():
        onehot = (seg == iota_b).astype(f32)
        u = lax.dot_general(feat_ref[...], wu_ref[...], (((1,), (1,)), ((), ())),
                            preferred_element_type=f32)
        fvb = lax.dot_general(onehot, fv_s[...], (((1,), (0,)), ((), ())),
                              preferred_element_type=f32)
        s = jax.nn.sigmoid(u + fvb)
        e_row = lax.dot_general(we_ref[...], s, (((1,), (1,)), ((), ())),
                                preferred_element_type=f32)
        eexp = jnp.where(valid_row, jnp.exp(e_row), 0.0)
        eexp_s[pl.ds(i, 1), :] = eexp
        den_s[...] += lax.dot_general(eexp, onehot, (((1,), (0,)), ((), ())),
                                      preferred_element_type=f32)

    @pl.when(p == 1)
    def _phase1():
        oh_bf = (seg == iota_b).astype(jnp.bfloat16)
        eexp_col = jnp.transpose(eexp_s[pl.ds(i, 1), :], (1, 0))
        valid_col = (i * BLK + lax.broadcasted_iota(jnp.int32, (BLK, 1), 0)) < n
        featn = jnp.where(valid_col, feat_ref[...] * eexp_col, 0.0)
        contrib = lax.dot_general(oh_bf, featn.astype(jnp.bfloat16),
                                  (((0,), (0,)), ((), ())),
                                  preferred_element_type=f32)

        @pl.when(i == 0)
        def _():
            out_ref[...] = contrib

        @pl.when(i > 0)
        def _():
            out_ref[...] += contrib

        @pl.when(i == nb - 1)
        def _():
            den = den_s[...]
            inv_col = jnp.transpose(jnp.where(den > 0, 1.0 / den, 0.0), (1, 0))
            acc = out_ref[...] * inv_col
            out_ref[...] = jnp.where(acc > 0, acc, pw_ref[...] * acc)


def _tc_main(feat, seg_pad, gathered, W_u, W_v, b_v, W_e, prelu_w):
    n, d = feat.shape
    h = W_u.shape[0]
    b = gathered.shape[0]
    nb = seg_pad.shape[0] // BLK
    nb_pad = ((nb + 7) // 8) * 8

    grid = (2, nb)
    body = functools.partial(_tc_body, n=n, b=b, nb=nb)
    return pl.pallas_call(
        body,
        grid=grid,
        in_specs=[
            pl.BlockSpec((BLK, d), lambda p, i: (i, 0)),
            pl.BlockSpec((BLK, 1), lambda p, i: (i, 0)),
            pl.BlockSpec((b, d), lambda p, i: (0, 0)),
            pl.BlockSpec((h, d), lambda p, i: (0, 0)),
            pl.BlockSpec((h, d), lambda p, i: (0, 0)),
            pl.BlockSpec((1, h), lambda p, i: (0, 0)),
            pl.BlockSpec((1, h), lambda p, i: (0, 0)),
            pl.BlockSpec((1, d), lambda p, i: (0, 0)),
        ],
        out_specs=pl.BlockSpec((b, d), lambda p, i: (0, 0)),
        out_shape=jax.ShapeDtypeStruct((b, d), jnp.float32),
        scratch_shapes=[
            pltpu.VMEM((b, h), jnp.float32),
            pltpu.VMEM((nb_pad, BLK), jnp.float32),
            pltpu.VMEM((1, b), jnp.float32),
        ],
        compiler_params=pltpu.CompilerParams(
            dimension_semantics=("arbitrary", "arbitrary"),
        ),
    )(feat, seg_pad, gathered, W_u, W_v, b_v, W_e, prelu_w)


def kernel(feat, segment_ids, last_nodes, W_u, W_v, b_v, W_e, prelu_w):
    n, d = feat.shape
    h = W_u.shape[0]
    b = last_nodes.shape[0]
    nb = -(-n // BLK)
    np_ = nb * BLK

    seg32 = segment_ids.astype(jnp.int32)
    seg_pad = jnp.full((np_,), b, jnp.int32).at[:n].set(seg32).reshape(np_, 1)
    idx32 = last_nodes.astype(jnp.int32)

    gathered = _sc_gather(feat, idx32)
    return _tc_main(feat, seg_pad, gathered,
                    W_u, W_v,
                    b_v.reshape(1, h).astype(jnp.float32),
                    W_e, prelu_w.reshape(1, d).astype(jnp.float32))


# speedup vs baseline: 1.9102x; 1.0819x over previous
import functools

import jax
import jax.numpy as jnp
from jax import lax
from jax.experimental import pallas as pl
from jax.experimental.pallas import tpu as pltpu
from jax.experimental.pallas import tpu_sc as plsc

BLK = 2048


def _sc_gather(feat, idx32):
    b = idx32.shape[0]
    d = feat.shape[1]
    window = 128
    mesh = plsc.VectorSubcoreMesh(core_axis_name="core", subcore_axis_name="subcore")
    indices = idx32.reshape((1, b))

    @functools.partial(
        pl.kernel,
        out_type=jax.ShapeDtypeStruct((b, d), feat.dtype),
        mesh=mesh,
    )
    def kern(x_hbm, i_hbm, o_hbm):
        def body(i_vmem, o_vmem):
            pltpu.sync_copy(x_hbm.at[i_vmem.at[0]], o_vmem)

        pltpu.emit_pipeline(
            body,
            grid=(b // window,),
            in_specs=[pl.BlockSpec((1, window), index_map=lambda i: (0, i))],
            out_specs=[pl.BlockSpec((window, d), index_map=lambda i: (i, 0))],
            core_axis_name="subcore",
            dimension_semantics=(pltpu.PARALLEL,),
        )(i_hbm, o_hbm)

    return kern(feat, indices)


def _tc_body(feat_ref, seg_ref, gat_ref, wu_ref, wv_ref, bv_ref, we_ref, pw_ref,
             out_ref, fv_s, eexp_s, den_s, *, n, b, nb):
    p = pl.program_id(0)
    i = pl.program_id(1)
    f32 = jnp.float32

    @pl.when(jnp.logical_and(p == 0, i == 0))
    def _prologue():
        fv = lax.dot_general(gat_ref[...], wv_ref[...], (((1,), (1,)), ((), ())),
                             preferred_element_type=f32)
        fv_s[...] = fv + bv_ref[...]
        den_s[...] = jnp.zeros_like(den_s)

    seg = seg_ref[...]
    iota_b = lax.broadcasted_iota(jnp.int32, (BLK, b), 1)
    valid_row = (i * BLK + lax.broadcasted_iota(jnp.int32, (1, BLK), 1)) < n

    @pl.when(p == 0)
    def _phase0():
        onehot = (seg == iota_b).astype(f32)
        u = lax.dot_general(feat_ref[...], wu_ref[...], (((1,), (1,)), ((), ())),
                            preferred_element_type=f32)
        fvb = lax.dot_general(onehot, fv_s[...], (((1,), (0,)), ((), ())),
                              preferred_element_type=f32)
        s = jax.nn.sigmoid(u + fvb)
        e_row = lax.dot_general(we_ref[...], s, (((1,), (1,)), ((), ())),
                                preferred_element_type=f32)
        eexp = jnp.where(valid_row, jnp.exp(e_row), 0.0)
        eexp_s[pl.ds(i, 1), :] = eexp
        den_s[...] += lax.dot_general(eexp, onehot, (((1,), (0,)), ((), ())),
                                      preferred_element_type=f32)

    @pl.when(p == 1)
    def _phase1():
        oh_bf = (seg == iota_b).astype(jnp.bfloat16)
        eexp_col = jnp.transpose(eexp_s[pl.ds(i, 1), :], (1, 0))
        valid_col = (i * BLK + lax.broadcasted_iota(jnp.int32, (BLK, 1), 0)) < n
        featn = jnp.where(valid_col, feat_ref[...] * eexp_col, 0.0)
        contrib = lax.dot_general(oh_bf, featn.astype(jnp.bfloat16),
                                  (((0,), (0,)), ((), ())),
                                  preferred_element_type=f32)

        @pl.when(i == 0)
        def _():
            out_ref[...] = contrib

        @pl.when(i > 0)
        def _():
            out_ref[...] += contrib

        @pl.when(i == nb - 1)
        def _():
            den = den_s[...]
            inv_col = jnp.transpose(jnp.where(den > 0, 1.0 / den, 0.0), (1, 0))
            acc = out_ref[...] * inv_col
            out_ref[...] = jnp.where(acc > 0, acc, pw_ref[...] * acc)


def _tc_main(feat, seg_pad, gathered, W_u, W_v, b_v, W_e, prelu_w):
    n, d = feat.shape
    h = W_u.shape[0]
    b = gathered.shape[0]
    nb = seg_pad.shape[0] // BLK
    nb_pad = ((nb + 7) // 8) * 8

    grid = (2, nb)
    body = functools.partial(_tc_body, n=n, b=b, nb=nb)
    return pl.pallas_call(
        body,
        grid=grid,
        in_specs=[
            pl.BlockSpec((BLK, d), lambda p, i: (i, 0)),
            pl.BlockSpec((BLK, 1), lambda p, i: (i, 0)),
            pl.BlockSpec((b, d), lambda p, i: (0, 0)),
            pl.BlockSpec((h, d), lambda p, i: (0, 0)),
            pl.BlockSpec((h, d), lambda p, i: (0, 0)),
            pl.BlockSpec((1, h), lambda p, i: (0, 0)),
            pl.BlockSpec((1, h), lambda p, i: (0, 0)),
            pl.BlockSpec((1, d), lambda p, i: (0, 0)),
        ],
        out_specs=pl.BlockSpec((b, d), lambda p, i: (0, 0)),
        out_shape=jax.ShapeDtypeStruct((b, d), jnp.float32),
        scratch_shapes=[
            pltpu.VMEM((b, h), jnp.float32),
            pltpu.VMEM((nb_pad, BLK), jnp.float32),
            pltpu.VMEM((1, b), jnp.float32),
        ],
        compiler_params=pltpu.CompilerParams(
            dimension_semantics=("arbitrary", "arbitrary"),
        ),
    )(feat, seg_pad, gathered, W_u, W_v, b_v, W_e, prelu_w)


def kernel(feat, segment_ids, last_nodes, W_u, W_v, b_v, W_e, prelu_w):
    n, d = feat.shape
    h = W_u.shape[0]
    b = last_nodes.shape[0]
    nb = -(-n // BLK)
    np_ = nb * BLK

    seg32 = segment_ids.astype(jnp.int32)
    seg_pad = jnp.full((np_,), b, jnp.int32).at[:n].set(seg32).reshape(np_, 1)
    idx32 = last_nodes.astype(jnp.int32)

    gathered = _sc_gather(feat, idx32)
    return _tc_main(feat, seg_pad, gathered,
                    W_u, W_v,
                    b_v.reshape(1, h).astype(jnp.float32),
                    W_e, prelu_w.reshape(1, d).astype(jnp.float32))


# speedup vs baseline: 1.9554x; 1.0237x over previous
import functools

import jax
import jax.numpy as jnp
from jax import lax
from jax.experimental import pallas as pl
from jax.experimental.pallas import tpu as pltpu
from jax.experimental.pallas import tpu_sc as plsc

BLK = 2560


def _sc_gather(feat, idx32):
    b = idx32.shape[0]
    d = feat.shape[1]
    window = 128
    mesh = plsc.VectorSubcoreMesh(core_axis_name="core", subcore_axis_name="subcore")
    indices = idx32.reshape((1, b))

    @functools.partial(
        pl.kernel,
        out_type=jax.ShapeDtypeStruct((b, d), feat.dtype),
        mesh=mesh,
    )
    def kern(x_hbm, i_hbm, o_hbm):
        def body(i_vmem, o_vmem):
            pltpu.sync_copy(x_hbm.at[i_vmem.at[0]], o_vmem)

        pltpu.emit_pipeline(
            body,
            grid=(b // window,),
            in_specs=[pl.BlockSpec((1, window), index_map=lambda i: (0, i))],
            out_specs=[pl.BlockSpec((window, d), index_map=lambda i: (i, 0))],
            core_axis_name="subcore",
            dimension_semantics=(pltpu.PARALLEL,),
        )(i_hbm, o_hbm)

    return kern(feat, indices)


def _tc_body(feat_ref, seg_ref, gat_ref, wu_ref, wv_ref, bv_ref, we_ref, pw_ref,
             out_ref, fv_s, eexp_s, den_s, *, n, b, nb):
    p = pl.program_id(0)
    i = pl.program_id(1)
    f32 = jnp.float32

    @pl.when(jnp.logical_and(p == 0, i == 0))
    def _prologue():
        fv = lax.dot_general(gat_ref[...], wv_ref[...], (((1,), (1,)), ((), ())),
                             preferred_element_type=f32)
        fv_s[...] = fv + bv_ref[...]
        den_s[...] = jnp.zeros_like(den_s)

    seg = seg_ref[...]
    iota_b = lax.broadcasted_iota(jnp.int32, (BLK, b), 1)
    valid_row = (i * BLK + lax.broadcasted_iota(jnp.int32, (1, BLK), 1)) < n

    @pl.when(p == 0)
    def _phase0():
        onehot = (seg == iota_b).astype(f32)
        u = lax.dot_general(feat_ref[...], wu_ref[...], (((1,), (1,)), ((), ())),
                            preferred_element_type=f32)
        fvb = lax.dot_general(onehot, fv_s[...], (((1,), (0,)), ((), ())),
                              preferred_element_type=f32)
        s = jax.nn.sigmoid(u + fvb)
        e_row = lax.dot_general(we_ref[...], s, (((1,), (1,)), ((), ())),
                                preferred_element_type=f32)
        eexp = jnp.where(valid_row, jnp.exp(e_row), 0.0)
        eexp_s[pl.ds(i, 1), :] = eexp
        den_s[...] += lax.dot_general(eexp, onehot, (((1,), (0,)), ((), ())),
                                      preferred_element_type=f32)

    @pl.when(p == 1)
    def _phase1():
        oh_bf = (seg == iota_b).astype(jnp.bfloat16)
        eexp_col = jnp.transpose(eexp_s[pl.ds(i, 1), :], (1, 0))
        valid_col = (i * BLK + lax.broadcasted_iota(jnp.int32, (BLK, 1), 0)) < n
        featn = jnp.where(valid_col, feat_ref[...] * eexp_col, 0.0)
        contrib = lax.dot_general(oh_bf, featn.astype(jnp.bfloat16),
                                  (((0,), (0,)), ((), ())),
                                  preferred_element_type=f32)

        @pl.when(i == 0)
        def _():
            out_ref[...] = contrib

        @pl.when(i > 0)
        def _():
            out_ref[...] += contrib

        @pl.when(i == nb - 1)
        def _():
            den = den_s[...]
            inv_col = jnp.transpose(jnp.where(den > 0, 1.0 / den, 0.0), (1, 0))
            acc = out_ref[...] * inv_col
            out_ref[...] = jnp.where(acc > 0, acc, pw_ref[...] * acc)


def _tc_main(feat, seg_pad, gathered, W_u, W_v, b_v, W_e, prelu_w):
    n, d = feat.shape
    h = W_u.shape[0]
    b = gathered.shape[0]
    nb = seg_pad.shape[0] // BLK
    nb_pad = ((nb + 7) // 8) * 8

    grid = (2, nb)
    body = functools.partial(_tc_body, n=n, b=b, nb=nb)
    return pl.pallas_call(
        body,
        grid=grid,
        in_specs=[
            pl.BlockSpec((BLK, d), lambda p, i: (i, 0)),
            pl.BlockSpec((BLK, 1), lambda p, i: (i, 0)),
            pl.BlockSpec((b, d), lambda p, i: (0, 0)),
            pl.BlockSpec((h, d), lambda p, i: (0, 0)),
            pl.BlockSpec((h, d), lambda p, i: (0, 0)),
            pl.BlockSpec((1, h), lambda p, i: (0, 0)),
            pl.BlockSpec((1, h), lambda p, i: (0, 0)),
            pl.BlockSpec((1, d), lambda p, i: (0, 0)),
        ],
        out_specs=pl.BlockSpec((b, d), lambda p, i: (0, 0)),
        out_shape=jax.ShapeDtypeStruct((b, d), jnp.float32),
        scratch_shapes=[
            pltpu.VMEM((b, h), jnp.float32),
            pltpu.VMEM((nb_pad, BLK), jnp.float32),
            pltpu.VMEM((1, b), jnp.float32),
        ],
        compiler_params=pltpu.CompilerParams(
            dimension_semantics=("arbitrary", "arbitrary"),
        ),
    )(feat, seg_pad, gathered, W_u, W_v, b_v, W_e, prelu_w)


def kernel(feat, segment_ids, last_nodes, W_u, W_v, b_v, W_e, prelu_w):
    n, d = feat.shape
    h = W_u.shape[0]
    b = last_nodes.shape[0]
    nb = -(-n // BLK)
    np_ = nb * BLK

    seg32 = segment_ids.astype(jnp.int32)
    seg_pad = jnp.full((np_,), b, jnp.int32).at[:n].set(seg32).reshape(np_, 1)
    idx32 = last_nodes.astype(jnp.int32)

    gathered = _sc_gather(feat, idx32)
    return _tc_main(feat, seg_pad, gathered,
                    W_u, W_v,
                    b_v.reshape(1, h).astype(jnp.float32),
                    W_e, prelu_w.reshape(1, d).astype(jnp.float32))
